# baseline (speedup 1.0000x reference)
import jax
import jax.numpy as jnp
from jax import lax
from jax.experimental import pallas as pl
from jax.experimental.pallas import tpu as pltpu
from jax.experimental.pallas import tpu_sc as plsc

N_NODES = 10000
D = 128
DH = D // 2

NC = 2
NS = 16
CHUNK = 128
EDGES_PAD = 327680
CHUNKS_TOTAL = EDGES_PAD // CHUNK
SCAT_CHUNKS = CHUNKS_TOTAL // NS
CNT_CHUNKS = CHUNKS_TOTAL // (NC * NS)
ACC_ROWS = 10240
ROWS_PER_TILE = ACC_ROWS // NS
DUMMY = N_NODES
CNT_W = 16

_mesh = plsc.VectorSubcoreMesh(core_axis_name="c", subcore_axis_name="s")


def _sc_scatter_body(ya_hbm, yb_hbm, src_hbm, dst_hbm, p_hbm,
                     srcv, dstv, rows0, rows1, rows2, rows3, acc,
                     g0, g1, g2, g3, s0, s1, s2, s3, zbuf):
    c = lax.axis_index("c")
    s = lax.axis_index("s")
    base = s * ROWS_PER_TILE
    rows = (rows0, rows1, rows2, rows3)
    gsem = (g0, g1, g2, g3)
    ssem = (s0, s1, s2, s3)

    zero16 = jnp.zeros((16,), jnp.float32)
    for i in range(zbuf.shape[0]):
        for j in range(DH // 16):
            zbuf[i, pl.ds(j * 16, 16)] = zero16

    pltpu.sync_copy(src_hbm.at[pl.ds(s * SCAT_CHUNKS, SCAT_CHUNKS)], srcv)
    pltpu.sync_copy(dst_hbm.at[pl.ds(s * SCAT_CHUNKS, SCAT_CHUNKS)], dstv)

    for t in range(ROWS_PER_TILE // zbuf.shape[0]):
        pltpu.sync_copy(zbuf, acc.at[pl.ds(base + t * zbuf.shape[0], zbuf.shape[0])])
    plsc.subcore_barrier()

    def run(y_hbm):
        def gather(j, b):
            pltpu.async_copy(y_hbm.at[srcv.at[j]], rows[b], gsem[b])

        def wait_gather(j, b):
            pltpu.make_async_copy(y_hbm.at[srcv.at[j]], rows[b], gsem[b]).wait()

        def scatter(j, b):
            pltpu.async_copy(rows[b], acc.at[dstv.at[j]], ssem[b], add=True)

        def wait_scatter(j, b):
            pltpu.make_async_copy(rows[b], acc.at[dstv.at[j]], ssem[b]).wait()

        gather(0, 0)
        gather(1, 1)
        for j in (0, 1):
            wait_gather(j, j)
            scatter(j, j)
            gather(j + 2, j + 2)

        def step(i, carry):
            for b in range(4):
                j = i * 4 + 2 + b
                jb = (2 + b) % 4
                nb = b
                wait_gather(j, jb)
                scatter(j, jb)
                wait_scatter(j - 2, nb)
                gather(j + 2, nb)
            return carry

        lax.fori_loop(0, (SCAT_CHUNKS - 4) // 4, step, 0)
        for j in (SCAT_CHUNKS - 2, SCAT_CHUNKS - 1):
            b = j % 4
            wait_gather(j, b)
            scatter(j, b)
        for j in range(SCAT_CHUNKS - 4, SCAT_CHUNKS):
            wait_scatter(j, j % 4)

    @pl.when(c == 0)
    def _():
        run(ya_hbm)

    @pl.when(c == 1)
    def _():
        run(yb_hbm)

    plsc.subcore_barrier()
    pltpu.sync_copy(acc.at[pl.ds(base, ROWS_PER_TILE)],
                    p_hbm.at[c, pl.ds(base, ROWS_PER_TILE)])


_sc_scatter = pl.kernel(
    _sc_scatter_body,
    out_type=[jax.ShapeDtypeStruct((NC, ACC_ROWS, DH), jnp.float32)],
    mesh=_mesh,
    scratch_types=(
        [pltpu.VMEM((SCAT_CHUNKS, CHUNK), jnp.int32)] * 2
        + [pltpu.VMEM((CHUNK, DH), jnp.float32)] * 4
        + [pltpu.VMEM_SHARED((ACC_ROWS, DH), jnp.float32)]
        + [pltpu.SemaphoreType.DMA] * 8
        + [pltpu.VMEM((16, DH), jnp.float32)]
    ),
    compiler_params=pltpu.CompilerParams(use_tc_tiling_on_sc=False),
)


def _sc_counts_body(dst_hbm, cnt_hbm, dstv, onesv, cacc, zcnt):
    c = lax.axis_index("c")
    s = lax.axis_index("s")
    wid = s * NC + c
    base = s * ROWS_PER_TILE

    zero16 = jnp.zeros((16,), jnp.float32)
    one16 = jnp.ones((16,), jnp.float32)
    for i in range(CHUNK):
        onesv[i, :] = one16
    for i in range(zcnt.shape[0]):
        zcnt[i, :] = zero16

    pltpu.sync_copy(dst_hbm.at[pl.ds(wid * CNT_CHUNKS, CNT_CHUNKS)], dstv)
    for t in range(ROWS_PER_TILE // zcnt.shape[0]):
        pltpu.sync_copy(zcnt, cacc.at[pl.ds(base + t * zcnt.shape[0], zcnt.shape[0])])
    plsc.subcore_barrier()

    def step(j, carry):
        pltpu.sync_copy(onesv, cacc.at[dstv.at[j]], add=True)
        return carry

    lax.fori_loop(0, CNT_CHUNKS, step, 0)
    plsc.subcore_barrier()

    pltpu.sync_copy(cacc.at[pl.ds(base, ROWS_PER_TILE)],
                    cnt_hbm.at[c, pl.ds(base, ROWS_PER_TILE)])


_sc_counts = pl.kernel(
    _sc_counts_body,
    out_type=[jax.ShapeDtypeStruct((NC, ACC_ROWS, CNT_W), jnp.float32)],
    mesh=_mesh,
    scratch_types=[
        pltpu.VMEM((CNT_CHUNKS, CHUNK), jnp.int32),
        pltpu.VMEM((CHUNK, CNT_W), jnp.float32),
        pltpu.VMEM_SHARED((ACC_ROWS, CNT_W), jnp.float32),
        pltpu.VMEM((64, CNT_W), jnp.float32),
    ],
    compiler_params=pltpu.CompilerParams(use_tc_tiling_on_sc=False),
)


_BR = 1000
_GRID = (N_NODES // _BR,)


def _row_spec(w=D):
    return pl.BlockSpec((_BR, w), lambda i: (i, 0))


def _full_spec(shape):
    return pl.BlockSpec(shape, lambda i: (0,) * len(shape))


def _tc1_body(x_ref, wl_ref, wr_ref, b_ref, ya_ref, yb_ref, z_ref):
    x = x_ref[...]
    y = jnp.dot(x, wl_ref[...], preferred_element_type=jnp.float32)
    ya_ref[...] = y[:, :DH]
    yb_ref[...] = y[:, DH:]
    z_ref[...] = jnp.dot(x, wr_ref[...], preferred_element_type=jnp.float32) + b_ref[...]


def _tc2_body(pa_ref, pb_ref, ca_ref, cb_ref, z1_ref, wl_ref, wr_ref, b_ref,
              ya_ref, yb_ref, z_ref):
    inv = 1.0 / jnp.clip(ca_ref[...][:, 0:1] + cb_ref[...][:, 0:1], 1.0, None)
    mean = jnp.concatenate([pa_ref[...], pb_ref[...]], axis=1) * inv
    h = jnp.maximum(mean + z1_ref[...], 0.0)
    y = jnp.dot(h, wl_ref[...], preferred_element_type=jnp.float32)
    ya_ref[...] = y[:, :DH]
    yb_ref[...] = y[:, DH:]
    z_ref[...] = jnp.dot(h, wr_ref[...], preferred_element_type=jnp.float32) + b_ref[...]


def _tc3_body(pa_ref, pb_ref, ca_ref, cb_ref, z2_ref, o_ref):
    inv = 1.0 / jnp.clip(ca_ref[...][:, 0:1] + cb_ref[...][:, 0:1], 1.0, None)
    mean = jnp.concatenate([pa_ref[...], pb_ref[...]], axis=1) * inv
    o_ref[...] = mean + z2_ref[...]


_tc1 = pl.pallas_call(
    _tc1_body,
    grid=_GRID,
    in_specs=[_row_spec(), _full_spec((D, D)), _full_spec((D, D)),
              _full_spec((1, D))],
    out_specs=[_row_spec(DH), _row_spec(DH), _row_spec()],
    out_shape=[jax.ShapeDtypeStruct((N_NODES, DH), jnp.float32),
               jax.ShapeDtypeStruct((N_NODES, DH), jnp.float32),
               jax.ShapeDtypeStruct((N_NODES, D), jnp.float32)],
)

_tc2 = pl.pallas_call(
    _tc2_body,
    grid=_GRID,
    in_specs=[_row_spec(DH), _row_spec(DH), _row_spec(CNT_W), _row_spec(CNT_W),
              _row_spec(), _full_spec((D, D)), _full_spec((D, D)),
              _full_spec((1, D))],
    out_specs=[_row_spec(DH), _row_spec(DH), _row_spec()],
    out_shape=[jax.ShapeDtypeStruct((N_NODES, DH), jnp.float32),
               jax.ShapeDtypeStruct((N_NODES, DH), jnp.float32),
               jax.ShapeDtypeStruct((N_NODES, D), jnp.float32)],
)

_tc3 = pl.pallas_call(
    _tc3_body,
    grid=_GRID,
    in_specs=[_row_spec(DH), _row_spec(DH), _row_spec(CNT_W), _row_spec(CNT_W),
              _row_spec()],
    out_specs=_row_spec(),
    out_shape=jax.ShapeDtypeStruct((N_NODES, D), jnp.float32),
)


@jax.jit
def kernel(x, edge_index, W1_l, b1, W1_r, W2_l, b2, W2_r):
    n_edges = edge_index.shape[1]
    pad = EDGES_PAD - n_edges
    src = jnp.concatenate(
        [edge_index[0].astype(jnp.int32), jnp.zeros((pad,), jnp.int32)]
    ).reshape(-1, CHUNK)
    dst = jnp.concatenate(
        [edge_index[1].astype(jnp.int32), jnp.full((pad,), DUMMY, jnp.int32)]
    ).reshape(-1, CHUNK)

    (cnt,) = _sc_counts(dst)
    ya1, yb1, z1 = _tc1(x, W1_l.T, W1_r.T, b1[None, :])
    (p1,) = _sc_scatter(ya1, yb1, src, dst)
    ya2, yb2, z2 = _tc2(p1[0], p1[1], cnt[0], cnt[1], z1,
                        W2_l.T, W2_r.T, b2[None, :])
    (p2,) = _sc_scatter(ya2, yb2, src, dst)
    out = _tc3(p2[0], p2[1], cnt[0], cnt[1], z2)
    return out


# speedup vs baseline: 1.7254x; 1.7254x over previous
import jax
import jax.numpy as jnp
from jax import lax
from jax.experimental import pallas as pl
from jax.experimental.pallas import tpu as pltpu
from jax.experimental.pallas import tpu_sc as plsc

N_NODES = 10000
D = 128
DH = D // 2

NC = 2
NS = 16
CHUNK = 128
EDGES_PAD = 327680
CHUNKS_TOTAL = EDGES_PAD // CHUNK
SCAT_CHUNKS = CHUNKS_TOTAL // NS
CNT_CHUNKS = CHUNKS_TOTAL // (NC * NS)
ACC_ROWS = 10240
ROWS_PER_TILE = ACC_ROWS // NS
DUMMY = N_NODES
CNT_W = 16

_mesh = plsc.VectorSubcoreMesh(core_axis_name="c", subcore_axis_name="s")


_YROWS_PER_TILE = N_NODES // NS


def _sc_scatter_body(ya_hbm, yb_hbm, eidx_hbm, p_hbm,
                     eidxv, rows0, rows1, srcj, dstj, yspm, acc,
                     sem0, sem1, zbuf):
    c = lax.axis_index("c")
    s = lax.axis_index("s")
    base = s * ROWS_PER_TILE

    zero16 = jnp.zeros((16,), jnp.float32)
    for i in range(zbuf.shape[0]):
        for j in range(DH // 16):
            zbuf[i, pl.ds(j * 16, 16)] = zero16

    pltpu.sync_copy(eidx_hbm.at[pl.ds(s * SCAT_CHUNKS, SCAT_CHUNKS)], eidxv)
    ybase = s * _YROWS_PER_TILE

    @pl.when(c == 0)
    def _():
        pltpu.sync_copy(ya_hbm.at[pl.ds(ybase, _YROWS_PER_TILE)],
                        yspm.at[pl.ds(ybase, _YROWS_PER_TILE)])

    @pl.when(c == 1)
    def _():
        pltpu.sync_copy(yb_hbm.at[pl.ds(ybase, _YROWS_PER_TILE)],
                        yspm.at[pl.ds(ybase, _YROWS_PER_TILE)])

    for t in range(ROWS_PER_TILE // zbuf.shape[0]):
        pltpu.sync_copy(zbuf, acc.at[pl.ds(base + t * zbuf.shape[0], zbuf.shape[0])])

    def unpack(j, b):
        for k in range(CHUNK // 16):
            v = eidxv[j, pl.ds(k * 16, 16)]
            srcj[b, pl.ds(k * 16, 16)] = lax.bitwise_and(v, 0xFFFF)
            dstj[b, pl.ds(k * 16, 16)] = lax.shift_right_logical(v, 16)

    plsc.subcore_barrier()

    for b in range(2):
        unpack(b, b)
        pltpu.async_copy(yspm.at[srcj.at[b]], (rows0, rows1)[b],
                         (sem0, sem1)[b])

    def step(i, carry):
        for b, (rows, sem) in enumerate(((rows0, sem0), (rows1, sem1))):
            j = i * 2 + b
            pltpu.make_async_copy(yspm.at[srcj.at[b]], rows, sem).wait()
            pltpu.sync_copy(rows, acc.at[dstj.at[b]], add=True)

            @pl.when(j + 2 < SCAT_CHUNKS)
            def _():
                unpack(j + 2, b)
                pltpu.async_copy(yspm.at[srcj.at[b]], rows, sem)
        return carry

    lax.fori_loop(0, SCAT_CHUNKS // 2, step, 0)

    plsc.subcore_barrier()
    pltpu.sync_copy(acc.at[pl.ds(base, ROWS_PER_TILE)],
                    p_hbm.at[c, pl.ds(base, ROWS_PER_TILE)])


_sc_scatter = pl.kernel(
    _sc_scatter_body,
    out_type=[jax.ShapeDtypeStruct((NC, ACC_ROWS, DH), jnp.float32)],
    mesh=_mesh,
    scratch_types=(
        [pltpu.VMEM((SCAT_CHUNKS, CHUNK), jnp.int32)]
        + [pltpu.VMEM((CHUNK, DH), jnp.float32)] * 2
        + [pltpu.VMEM((2, CHUNK), jnp.int32)] * 2
        + [pltpu.VMEM_SHARED((N_NODES, DH), jnp.float32)]
        + [pltpu.VMEM_SHARED((ACC_ROWS, DH), jnp.float32)]
        + [pltpu.SemaphoreType.DMA] * 2
        + [pltpu.VMEM((16, DH), jnp.float32)]
    ),
    compiler_params=pltpu.CompilerParams(use_tc_tiling_on_sc=False),
)


def _sc_counts_body(eidx_hbm, cnt_hbm, dstv, onesv, cacc, zcnt):
    c = lax.axis_index("c")
    s = lax.axis_index("s")
    wid = s * NC + c
    base = s * ROWS_PER_TILE

    zero16 = jnp.zeros((16,), jnp.float32)
    one16 = jnp.ones((16,), jnp.float32)
    for i in range(CHUNK):
        onesv[i, :] = one16
    for i in range(zcnt.shape[0]):
        zcnt[i, :] = zero16

    pltpu.sync_copy(eidx_hbm.at[pl.ds(wid * CNT_CHUNKS, CNT_CHUNKS)], dstv)

    def unpack(i, carry):
        for k in range(CHUNK // 16):
            dstv[i, pl.ds(k * 16, 16)] = lax.shift_right_logical(
                dstv[i, pl.ds(k * 16, 16)], 16)
        return carry

    lax.fori_loop(0, CNT_CHUNKS, unpack, 0)
    for t in range(ROWS_PER_TILE // zcnt.shape[0]):
        pltpu.sync_copy(zcnt, cacc.at[pl.ds(base + t * zcnt.shape[0], zcnt.shape[0])])
    plsc.subcore_barrier()

    def step(j, carry):
        pltpu.sync_copy(onesv, cacc.at[dstv.at[j]], add=True)
        return carry

    lax.fori_loop(0, CNT_CHUNKS, step, 0)
    plsc.subcore_barrier()

    pltpu.sync_copy(cacc.at[pl.ds(base, ROWS_PER_TILE)],
                    cnt_hbm.at[c, pl.ds(base, ROWS_PER_TILE)])


_sc_counts = pl.kernel(
    _sc_counts_body,
    out_type=[jax.ShapeDtypeStruct((NC, ACC_ROWS, CNT_W), jnp.float32)],
    mesh=_mesh,
    scratch_types=[
        pltpu.VMEM((CNT_CHUNKS, CHUNK), jnp.int32),
        pltpu.VMEM((CHUNK, CNT_W), jnp.float32),
        pltpu.VMEM_SHARED((ACC_ROWS, CNT_W), jnp.float32),
        pltpu.VMEM((64, CNT_W), jnp.float32),
    ],
    compiler_params=pltpu.CompilerParams(use_tc_tiling_on_sc=False),
)


_BR = 1000
_GRID = (N_NODES // _BR,)


def _row_spec(w=D):
    return pl.BlockSpec((_BR, w), lambda i: (i, 0))


def _full_spec(shape):
    return pl.BlockSpec(shape, lambda i: (0,) * len(shape))


def _tc1_body(x_ref, wl_ref, wr_ref, b_ref, ya_ref, yb_ref, z_ref):
    x = x_ref[...]
    y = jnp.dot(x, wl_ref[...], preferred_element_type=jnp.float32)
    ya_ref[...] = y[:, :DH]
    yb_ref[...] = y[:, DH:]
    z_ref[...] = jnp.dot(x, wr_ref[...], preferred_element_type=jnp.float32) + b_ref[...]


def _tc2_body(pa_ref, pb_ref, ca_ref, cb_ref, z1_ref, wl_ref, wr_ref, b_ref,
              ya_ref, yb_ref, z_ref):
    inv = 1.0 / jnp.clip(ca_ref[...][:, 0:1] + cb_ref[...][:, 0:1], 1.0, None)
    mean = jnp.concatenate([pa_ref[...], pb_ref[...]], axis=1) * inv
    h = jnp.maximum(mean + z1_ref[...], 0.0)
    y = jnp.dot(h, wl_ref[...], preferred_element_type=jnp.float32)
    ya_ref[...] = y[:, :DH]
    yb_ref[...] = y[:, DH:]
    z_ref[...] = jnp.dot(h, wr_ref[...], preferred_element_type=jnp.float32) + b_ref[...]


def _tc3_body(pa_ref, pb_ref, ca_ref, cb_ref, z2_ref, o_ref):
    inv = 1.0 / jnp.clip(ca_ref[...][:, 0:1] + cb_ref[...][:, 0:1], 1.0, None)
    mean = jnp.concatenate([pa_ref[...], pb_ref[...]], axis=1) * inv
    o_ref[...] = mean + z2_ref[...]


_tc1 = pl.pallas_call(
    _tc1_body,
    grid=_GRID,
    in_specs=[_row_spec(), _full_spec((D, D)), _full_spec((D, D)),
              _full_spec((1, D))],
    out_specs=[_row_spec(DH), _row_spec(DH), _row_spec()],
    out_shape=[jax.ShapeDtypeStruct((N_NODES, DH), jnp.float32),
               jax.ShapeDtypeStruct((N_NODES, DH), jnp.float32),
               jax.ShapeDtypeStruct((N_NODES, D), jnp.float32)],
)

_tc2 = pl.pallas_call(
    _tc2_body,
    grid=_GRID,
    in_specs=[_row_spec(DH), _row_spec(DH), _row_spec(CNT_W), _row_spec(CNT_W),
              _row_spec(), _full_spec((D, D)), _full_spec((D, D)),
              _full_spec((1, D))],
    out_specs=[_row_spec(DH), _row_spec(DH), _row_spec()],
    out_shape=[jax.ShapeDtypeStruct((N_NODES, DH), jnp.float32),
               jax.ShapeDtypeStruct((N_NODES, DH), jnp.float32),
               jax.ShapeDtypeStruct((N_NODES, D), jnp.float32)],
)

_tc3 = pl.pallas_call(
    _tc3_body,
    grid=_GRID,
    in_specs=[_row_spec(DH), _row_spec(DH), _row_spec(CNT_W), _row_spec(CNT_W),
              _row_spec()],
    out_specs=_row_spec(),
    out_shape=jax.ShapeDtypeStruct((N_NODES, D), jnp.float32),
)


@jax.jit
def kernel(x, edge_index, W1_l, b1, W1_r, W2_l, b2, W2_r):
    n_edges = edge_index.shape[1]
    pad = EDGES_PAD - n_edges
    src = jnp.concatenate(
        [edge_index[0].astype(jnp.int32), jnp.zeros((pad,), jnp.int32)])
    dst = jnp.concatenate(
        [edge_index[1].astype(jnp.int32), jnp.full((pad,), DUMMY, jnp.int32)])
    eidx = (src | (dst << 16)).reshape(-1, CHUNK)

    (cnt,) = _sc_counts(eidx)
    ya1, yb1, z1 = _tc1(x, W1_l.T, W1_r.T, b1[None, :])
    (p1,) = _sc_scatter(ya1, yb1, eidx)
    ya2, yb2, z2 = _tc2(p1[0], p1[1], cnt[0], cnt[1], z1,
                        W2_l.T, W2_r.T, b2[None, :])
    (p2,) = _sc_scatter(ya2, yb2, eidx)
    out = _tc3(p2[0], p2[1], cnt[0], cnt[1], z2)
    return out


# speedup vs baseline: 1.7842x; 1.0340x over previous
import jax
import jax.numpy as jnp
from jax import lax
from jax.experimental import pallas as pl
from jax.experimental.pallas import tpu as pltpu
from jax.experimental.pallas import tpu_sc as plsc

N_NODES = 10000
D = 128
DH = D // 2

NC = 2
NS = 16
CHUNK = 128
EDGES_PAD = 327680
CHUNKS_TOTAL = EDGES_PAD // CHUNK
SCAT_CHUNKS = CHUNKS_TOTAL // NS
CNT_CHUNKS = CHUNKS_TOTAL // (NC * NS)
ACC_ROWS = 10240
ROWS_PER_TILE = ACC_ROWS // NS
DUMMY = N_NODES
CNT_W = 16

_mesh = plsc.VectorSubcoreMesh(core_axis_name="c", subcore_axis_name="s")


_YROWS_PER_TILE = N_NODES // NS


def _sc_scatter_body(ya_hbm, yb_hbm, eidx_hbm, p_hbm,
                     eidxv, rows0, rows1, srcj, dstj, yspm, acc,
                     sem0, sem1, zbuf):
    c = lax.axis_index("c")
    s = lax.axis_index("s")
    base = s * ROWS_PER_TILE

    zero16 = jnp.zeros((16,), jnp.float32)
    for i in range(zbuf.shape[0]):
        for j in range(DH // 16):
            zbuf[i, pl.ds(j * 16, 16)] = zero16

    pltpu.sync_copy(eidx_hbm.at[pl.ds(s * SCAT_CHUNKS, SCAT_CHUNKS)], eidxv)
    ybase = s * _YROWS_PER_TILE

    @pl.when(c == 0)
    def _():
        pltpu.sync_copy(ya_hbm.at[pl.ds(ybase, _YROWS_PER_TILE)],
                        yspm.at[pl.ds(ybase, _YROWS_PER_TILE)])

    @pl.when(c == 1)
    def _():
        pltpu.sync_copy(yb_hbm.at[pl.ds(ybase, _YROWS_PER_TILE)],
                        yspm.at[pl.ds(ybase, _YROWS_PER_TILE)])

    for t in range(ROWS_PER_TILE // zbuf.shape[0]):
        pltpu.sync_copy(zbuf, acc.at[pl.ds(base + t * zbuf.shape[0], zbuf.shape[0])])

    def unpack(j, b):
        for k in range(CHUNK // 16):
            v = eidxv[j, pl.ds(k * 16, 16)]
            srcj[b, pl.ds(k * 16, 16)] = lax.bitwise_and(v, 0xFFFF)
            dstj[b, pl.ds(k * 16, 16)] = lax.shift_right_logical(v, 16)

    plsc.subcore_barrier()

    for b in range(2):
        unpack(b, b)
        pltpu.async_copy(yspm.at[srcj.at[b]], (rows0, rows1)[b],
                         (sem0, sem1)[b])

    def step(i, carry):
        for b, (rows, sem) in enumerate(((rows0, sem0), (rows1, sem1))):
            j = i * 2 + b
            pltpu.make_async_copy(yspm.at[srcj.at[b]], rows, sem).wait()
            pltpu.sync_copy(rows, acc.at[dstj.at[b]], add=True)

            @pl.when(j + 2 < SCAT_CHUNKS)
            def _():
                unpack(j + 2, b)
                pltpu.async_copy(yspm.at[srcj.at[b]], rows, sem)
        return carry

    lax.fori_loop(0, SCAT_CHUNKS // 2, step, 0)

    plsc.subcore_barrier()
    pltpu.sync_copy(acc.at[pl.ds(base, ROWS_PER_TILE)],
                    p_hbm.at[c, pl.ds(base, ROWS_PER_TILE)])


_sc_scatter = pl.kernel(
    _sc_scatter_body,
    out_type=[jax.ShapeDtypeStruct((NC, ACC_ROWS, DH), jnp.float32)],
    mesh=_mesh,
    scratch_types=(
        [pltpu.VMEM((SCAT_CHUNKS, CHUNK), jnp.int32)]
        + [pltpu.VMEM((CHUNK, DH), jnp.float32)] * 2
        + [pltpu.VMEM((2, CHUNK), jnp.int32)] * 2
        + [pltpu.VMEM_SHARED((N_NODES, DH), jnp.float32)]
        + [pltpu.VMEM_SHARED((ACC_ROWS, DH), jnp.float32)]
        + [pltpu.SemaphoreType.DMA] * 2
        + [pltpu.VMEM((16, DH), jnp.float32)]
    ),
    compiler_params=pltpu.CompilerParams(use_tc_tiling_on_sc=False),
)


def _sc_counts_body(eidx_hbm, cnt_hbm, dstv, onesv, cacc, zcnt):
    c = lax.axis_index("c")
    s = lax.axis_index("s")
    wid = s * NC + c
    base = s * ROWS_PER_TILE

    zero16 = jnp.zeros((16,), jnp.float32)
    one16 = jnp.ones((16,), jnp.float32)
    for i in range(CHUNK):
        onesv[i, :] = one16
    for i in range(zcnt.shape[0]):
        zcnt[i, :] = zero16

    pltpu.sync_copy(eidx_hbm.at[pl.ds(wid * CNT_CHUNKS, CNT_CHUNKS)], dstv)

    def unpack(i, carry):
        for k in range(CHUNK // 16):
            dstv[i, pl.ds(k * 16, 16)] = lax.shift_right_logical(
                dstv[i, pl.ds(k * 16, 16)], 16)
        return carry

    lax.fori_loop(0, CNT_CHUNKS, unpack, 0)
    for t in range(ROWS_PER_TILE // zcnt.shape[0]):
        pltpu.sync_copy(zcnt, cacc.at[pl.ds(base + t * zcnt.shape[0], zcnt.shape[0])])
    plsc.subcore_barrier()

    def step(j, carry):
        pltpu.sync_copy(onesv, cacc.at[dstv.at[j]], add=True)
        return carry

    lax.fori_loop(0, CNT_CHUNKS, step, 0)
    plsc.subcore_barrier()

    pltpu.sync_copy(cacc.at[pl.ds(base, ROWS_PER_TILE)],
                    cnt_hbm.at[c, pl.ds(base, ROWS_PER_TILE)])


_sc_counts = pl.kernel(
    _sc_counts_body,
    out_type=[jax.ShapeDtypeStruct((NC, ACC_ROWS, CNT_W), jnp.float32)],
    mesh=_mesh,
    scratch_types=[
        pltpu.VMEM((CNT_CHUNKS, CHUNK), jnp.int32),
        pltpu.VMEM((CHUNK, CNT_W), jnp.float32),
        pltpu.VMEM_SHARED((ACC_ROWS, CNT_W), jnp.float32),
        pltpu.VMEM((64, CNT_W), jnp.float32),
    ],
    compiler_params=pltpu.CompilerParams(use_tc_tiling_on_sc=False),
)


_BR = 1000
_GRID = (N_NODES // _BR,)


def _row_spec(w=D):
    return pl.BlockSpec((_BR, w), lambda i: (i, 0))


def _plane_spec(plane, w):
    return pl.BlockSpec((1, _BR, w), lambda i, p=plane: (p, i, 0))


def _full_spec(shape):
    return pl.BlockSpec(shape, lambda i: (0,) * len(shape))


def _tc1_body(x_ref, wl_ref, wr_ref, b_ref, ya_ref, yb_ref, z_ref):
    x = x_ref[...]
    y = jnp.dot(x, wl_ref[...], preferred_element_type=jnp.float32)
    ya_ref[...] = y[:, :DH]
    yb_ref[...] = y[:, DH:]
    z_ref[...] = jnp.dot(x, wr_ref[...], preferred_element_type=jnp.float32) + b_ref[...]


def _tc2_body(pa_ref, pb_ref, ca_ref, cb_ref, z1_ref, wl_ref, wr_ref, b_ref,
              ya_ref, yb_ref, z_ref):
    inv = 1.0 / jnp.clip(ca_ref[0][:, 0:1] + cb_ref[0][:, 0:1], 1.0, None)
    mean = jnp.concatenate([pa_ref[0], pb_ref[0]], axis=1) * inv
    h = jnp.maximum(mean + z1_ref[...], 0.0)
    y = jnp.dot(h, wl_ref[...], preferred_element_type=jnp.float32)
    ya_ref[...] = y[:, :DH]
    yb_ref[...] = y[:, DH:]
    z_ref[...] = jnp.dot(h, wr_ref[...], preferred_element_type=jnp.float32) + b_ref[...]


def _tc3_body(pa_ref, pb_ref, ca_ref, cb_ref, z2_ref, o_ref):
    inv = 1.0 / jnp.clip(ca_ref[0][:, 0:1] + cb_ref[0][:, 0:1], 1.0, None)
    mean = jnp.concatenate([pa_ref[0], pb_ref[0]], axis=1) * inv
    o_ref[...] = mean + z2_ref[...]


_tc1 = pl.pallas_call(
    _tc1_body,
    grid=_GRID,
    in_specs=[_row_spec(), _full_spec((D, D)), _full_spec((D, D)),
              _full_spec((1, D))],
    out_specs=[_row_spec(DH), _row_spec(DH), _row_spec()],
    out_shape=[jax.ShapeDtypeStruct((N_NODES, DH), jnp.float32),
               jax.ShapeDtypeStruct((N_NODES, DH), jnp.float32),
               jax.ShapeDtypeStruct((N_NODES, D), jnp.float32)],
)

_tc2 = pl.pallas_call(
    _tc2_body,
    grid=_GRID,
    in_specs=[_plane_spec(0, DH), _plane_spec(1, DH), _plane_spec(0, CNT_W),
              _plane_spec(1, CNT_W), _row_spec(), _full_spec((D, D)),
              _full_spec((D, D)), _full_spec((1, D))],
    out_specs=[_row_spec(DH), _row_spec(DH), _row_spec()],
    out_shape=[jax.ShapeDtypeStruct((N_NODES, DH), jnp.float32),
               jax.ShapeDtypeStruct((N_NODES, DH), jnp.float32),
               jax.ShapeDtypeStruct((N_NODES, D), jnp.float32)],
)

_tc3 = pl.pallas_call(
    _tc3_body,
    grid=_GRID,
    in_specs=[_plane_spec(0, DH), _plane_spec(1, DH), _plane_spec(0, CNT_W),
              _plane_spec(1, CNT_W), _row_spec()],
    out_specs=_row_spec(),
    out_shape=jax.ShapeDtypeStruct((N_NODES, D), jnp.float32),
)


@jax.jit
def kernel(x, edge_index, W1_l, b1, W1_r, W2_l, b2, W2_r):
    n_edges = edge_index.shape[1]
    pad = EDGES_PAD - n_edges
    src = jnp.concatenate(
        [edge_index[0].astype(jnp.int32), jnp.zeros((pad,), jnp.int32)])
    dst = jnp.concatenate(
        [edge_index[1].astype(jnp.int32), jnp.full((pad,), DUMMY, jnp.int32)])
    eidx = (src | (dst << 16)).reshape(-1, CHUNK)

    (cnt,) = _sc_counts(eidx)
    ya1, yb1, z1 = _tc1(x, W1_l.T, W1_r.T, b1[None, :])
    (p1,) = _sc_scatter(ya1, yb1, eidx)
    ya2, yb2, z2 = _tc2(p1, p1, cnt, cnt, z1, W2_l.T, W2_r.T, b2[None, :])
    (p2,) = _sc_scatter(ya2, yb2, eidx)
    out = _tc3(p2, p2, cnt, cnt, z2)
    return out


# speedup vs baseline: 2.1175x; 1.1868x over previous
import jax
import jax.numpy as jnp
from jax import lax
from jax.experimental import pallas as pl
from jax.experimental.pallas import tpu as pltpu
from jax.experimental.pallas import tpu_sc as plsc

N_NODES = 10000
D = 128
DH = D // 2

NC = 2
NS = 16
CHUNK = 128
EDGES_PAD = 327680
CHUNKS_TOTAL = EDGES_PAD // CHUNK
SCAT_CHUNKS = CHUNKS_TOTAL // NS
CNT_CHUNKS = CHUNKS_TOTAL // (NC * NS)
ACC_ROWS = 10240
ROWS_PER_TILE = ACC_ROWS // NS
DUMMY = N_NODES
CNT_W = 16

_mesh = plsc.VectorSubcoreMesh(core_axis_name="c", subcore_axis_name="s")


_YROWS_PER_TILE = N_NODES // NS


def _sc_scatter_body(ya_hbm, yb_hbm, eidx_hbm, p_hbm,
                     eidxv, rows0, rows1, rows2, srcj, dstj, yspm, acc,
                     g0, g1, g2, s0, s1, s2, zbuf):
    gsem = (g0, g1, g2)
    ssem = (s0, s1, s2)
    c = lax.axis_index("c")
    s = lax.axis_index("s")
    base = s * ROWS_PER_TILE

    zero16 = jnp.zeros((16,), jnp.float32)
    for i in range(zbuf.shape[0]):
        for j in range(DH // 16):
            zbuf[i, pl.ds(j * 16, 16)] = zero16

    pltpu.sync_copy(eidx_hbm.at[pl.ds(s * SCAT_CHUNKS, SCAT_CHUNKS)], eidxv)
    ybase = s * _YROWS_PER_TILE

    @pl.when(c == 0)
    def _():
        pltpu.sync_copy(ya_hbm.at[pl.ds(ybase, _YROWS_PER_TILE)],
                        yspm.at[pl.ds(ybase, _YROWS_PER_TILE)])

    @pl.when(c == 1)
    def _():
        pltpu.sync_copy(yb_hbm.at[pl.ds(ybase, _YROWS_PER_TILE)],
                        yspm.at[pl.ds(ybase, _YROWS_PER_TILE)])

    for t in range(ROWS_PER_TILE // zbuf.shape[0]):
        pltpu.sync_copy(zbuf, acc.at[pl.ds(base + t * zbuf.shape[0], zbuf.shape[0])])

    def unpack(j, b):
        for k in range(CHUNK // 16):
            v = eidxv[j, pl.ds(k * 16, 16)]
            srcj[b, pl.ds(k * 16, 16)] = lax.bitwise_and(v, 0xFFFF)
            dstj[b, pl.ds(k * 16, 16)] = lax.shift_right_logical(v, 16)

    plsc.subcore_barrier()

    rows = (rows0, rows1, rows2)

    def gather(j, b):
        pltpu.async_copy(yspm.at[srcj.at[b]], rows[b], gsem[b])

    def wait_gather(b):
        pltpu.make_async_copy(yspm.at[srcj.at[b]], rows[b], gsem[b]).wait()

    def scatter(b):
        pltpu.async_copy(rows[b], acc.at[dstj.at[b]], ssem[b], add=True)

    def wait_scatter(b):
        pltpu.make_async_copy(rows[b], acc.at[dstj.at[b]], ssem[b]).wait()

    for b in range(2):
        unpack(b, b)
        gather(b, b)
    wait_gather(0)
    scatter(0)
    unpack(2, 2)
    gather(2, 2)

    def step3(j, b):
        wait_gather(b)
        scatter(b)
        nb = (b + 2) % 3
        wait_scatter(nb)
        unpack(j + 2, nb)
        gather(j + 2, nb)

    step3(1, 1)

    def step(i, carry):
        for b in range(3):
            step3(i * 3 + 2 + b, (2 + b) % 3)
        return carry

    lax.fori_loop(0, (SCAT_CHUNKS - 4) // 3, step, 0)
    for j in (SCAT_CHUNKS - 2, SCAT_CHUNKS - 1):
        b = j % 3
        wait_gather(b)
        scatter(b)
    for j in range(SCAT_CHUNKS - 3, SCAT_CHUNKS):
        wait_scatter(j % 3)

    plsc.subcore_barrier()
    pltpu.sync_copy(acc.at[pl.ds(base, ROWS_PER_TILE)],
                    p_hbm.at[c, pl.ds(base, ROWS_PER_TILE)])


_sc_scatter = pl.kernel(
    _sc_scatter_body,
    out_type=[jax.ShapeDtypeStruct((NC, ACC_ROWS, DH), jnp.float32)],
    mesh=_mesh,
    scratch_types=(
        [pltpu.VMEM((SCAT_CHUNKS, CHUNK), jnp.int32)]
        + [pltpu.VMEM((CHUNK, DH), jnp.float32)] * 3
        + [pltpu.VMEM((3, CHUNK), jnp.int32)] * 2
        + [pltpu.VMEM_SHARED((N_NODES, DH), jnp.float32)]
        + [pltpu.VMEM_SHARED((ACC_ROWS, DH), jnp.float32)]
        + [pltpu.SemaphoreType.DMA] * 6
        + [pltpu.VMEM((16, DH), jnp.float32)]
    ),
    compiler_params=pltpu.CompilerParams(use_tc_tiling_on_sc=False),
)


def _sc_counts_body(eidx_hbm, cnt_hbm, dstv, onesv, cacc, zcnt):
    c = lax.axis_index("c")
    s = lax.axis_index("s")
    wid = s * NC + c
    base = s * ROWS_PER_TILE

    zero16 = jnp.zeros((16,), jnp.float32)
    one16 = jnp.ones((16,), jnp.float32)
    for i in range(CHUNK):
        onesv[i, :] = one16
    for i in range(zcnt.shape[0]):
        zcnt[i, :] = zero16

    pltpu.sync_copy(eidx_hbm.at[pl.ds(wid * CNT_CHUNKS, CNT_CHUNKS)], dstv)

    def unpack(i, carry):
        for k in range(CHUNK // 16):
            dstv[i, pl.ds(k * 16, 16)] = lax.shift_right_logical(
                dstv[i, pl.ds(k * 16, 16)], 16)
        return carry

    lax.fori_loop(0, CNT_CHUNKS, unpack, 0)
    for t in range(ROWS_PER_TILE // zcnt.shape[0]):
        pltpu.sync_copy(zcnt, cacc.at[pl.ds(base + t * zcnt.shape[0], zcnt.shape[0])])
    plsc.subcore_barrier()

    def step(j, carry):
        pltpu.sync_copy(onesv, cacc.at[dstv.at[j]], add=True)
        return carry

    lax.fori_loop(0, CNT_CHUNKS, step, 0)
    plsc.subcore_barrier()

    pltpu.sync_copy(cacc.at[pl.ds(base, ROWS_PER_TILE)],
                    cnt_hbm.at[c, pl.ds(base, ROWS_PER_TILE)])


_sc_counts = pl.kernel(
    _sc_counts_body,
    out_type=[jax.ShapeDtypeStruct((NC, ACC_ROWS, CNT_W), jnp.float32)],
    mesh=_mesh,
    scratch_types=[
        pltpu.VMEM((CNT_CHUNKS, CHUNK), jnp.int32),
        pltpu.VMEM((CHUNK, CNT_W), jnp.float32),
        pltpu.VMEM_SHARED((ACC_ROWS, CNT_W), jnp.float32),
        pltpu.VMEM((64, CNT_W), jnp.float32),
    ],
    compiler_params=pltpu.CompilerParams(use_tc_tiling_on_sc=False),
)


_BR = 1000
_GRID = (N_NODES // _BR,)


def _row_spec(w=D):
    return pl.BlockSpec((_BR, w), lambda i: (i, 0))


def _plane_spec(plane, w):
    return pl.BlockSpec((1, _BR, w), lambda i, p=plane: (p, i, 0))


def _full_spec(shape):
    return pl.BlockSpec(shape, lambda i: (0,) * len(shape))


def _tc1_body(x_ref, wl_ref, wr_ref, b_ref, ya_ref, yb_ref, z_ref):
    x = x_ref[...]
    y = jnp.dot(x, wl_ref[...], preferred_element_type=jnp.float32)
    ya_ref[...] = y[:, :DH]
    yb_ref[...] = y[:, DH:]
    z_ref[...] = jnp.dot(x, wr_ref[...], preferred_element_type=jnp.float32) + b_ref[...]


def _tc2_body(pa_ref, pb_ref, ca_ref, cb_ref, z1_ref, wl_ref, wr_ref, b_ref,
              ya_ref, yb_ref, z_ref):
    inv = 1.0 / jnp.clip(ca_ref[0][:, 0:1] + cb_ref[0][:, 0:1], 1.0, None)
    mean = jnp.concatenate([pa_ref[0], pb_ref[0]], axis=1) * inv
    h = jnp.maximum(mean + z1_ref[...], 0.0)
    y = jnp.dot(h, wl_ref[...], preferred_element_type=jnp.float32)
    ya_ref[...] = y[:, :DH]
    yb_ref[...] = y[:, DH:]
    z_ref[...] = jnp.dot(h, wr_ref[...], preferred_element_type=jnp.float32) + b_ref[...]


def _tc3_body(pa_ref, pb_ref, ca_ref, cb_ref, z2_ref, o_ref):
    inv = 1.0 / jnp.clip(ca_ref[0][:, 0:1] + cb_ref[0][:, 0:1], 1.0, None)
    mean = jnp.concatenate([pa_ref[0], pb_ref[0]], axis=1) * inv
    o_ref[...] = mean + z2_ref[...]


_tc1 = pl.pallas_call(
    _tc1_body,
    grid=_GRID,
    in_specs=[_row_spec(), _full_spec((D, D)), _full_spec((D, D)),
              _full_spec((1, D))],
    out_specs=[_row_spec(DH), _row_spec(DH), _row_spec()],
    out_shape=[jax.ShapeDtypeStruct((N_NODES, DH), jnp.float32),
               jax.ShapeDtypeStruct((N_NODES, DH), jnp.float32),
               jax.ShapeDtypeStruct((N_NODES, D), jnp.float32)],
)

_tc2 = pl.pallas_call(
    _tc2_body,
    grid=_GRID,
    in_specs=[_plane_spec(0, DH), _plane_spec(1, DH), _plane_spec(0, CNT_W),
              _plane_spec(1, CNT_W), _row_spec(), _full_spec((D, D)),
              _full_spec((D, D)), _full_spec((1, D))],
    out_specs=[_row_spec(DH), _row_spec(DH), _row_spec()],
    out_shape=[jax.ShapeDtypeStruct((N_NODES, DH), jnp.float32),
               jax.ShapeDtypeStruct((N_NODES, DH), jnp.float32),
               jax.ShapeDtypeStruct((N_NODES, D), jnp.float32)],
)

_tc3 = pl.pallas_call(
    _tc3_body,
    grid=_GRID,
    in_specs=[_plane_spec(0, DH), _plane_spec(1, DH), _plane_spec(0, CNT_W),
              _plane_spec(1, CNT_W), _row_spec()],
    out_specs=_row_spec(),
    out_shape=jax.ShapeDtypeStruct((N_NODES, D), jnp.float32),
)


@jax.jit
def kernel(x, edge_index, W1_l, b1, W1_r, W2_l, b2, W2_r):
    n_edges = edge_index.shape[1]
    pad = EDGES_PAD - n_edges
    src = jnp.concatenate(
        [edge_index[0].astype(jnp.int32), jnp.zeros((pad,), jnp.int32)])
    dst = jnp.concatenate(
        [edge_index[1].astype(jnp.int32), jnp.full((pad,), DUMMY, jnp.int32)])
    eidx = (src | (dst << 16)).reshape(-1, CHUNK)

    (cnt,) = _sc_counts(eidx)
    ya1, yb1, z1 = _tc1(x, W1_l.T, W1_r.T, b1[None, :])
    (p1,) = _sc_scatter(ya1, yb1, eidx)
    ya2, yb2, z2 = _tc2(p1, p1, cnt, cnt, z1, W2_l.T, W2_r.T, b2[None, :])
    (p2,) = _sc_scatter(ya2, yb2, eidx)
    out = _tc3(p2, p2, cnt, cnt, z2)
    return out


# speedup vs baseline: 2.1765x; 1.0279x over previous
import jax
import jax.numpy as jnp
from jax import lax
from jax.experimental import pallas as pl
from jax.experimental.pallas import tpu as pltpu
from jax.experimental.pallas import tpu_sc as plsc

N_NODES = 10000
D = 128
DH = D // 2

NC = 2
NS = 16
CHUNK = 128
EDGES_PAD = 327680
CHUNKS_TOTAL = EDGES_PAD // CHUNK
SCAT_CHUNKS = CHUNKS_TOTAL // NS
CNT_CHUNKS = CHUNKS_TOTAL // (NC * NS)
ACC_ROWS = 10240
ROWS_PER_TILE = ACC_ROWS // NS
DUMMY = N_NODES
CNT_W = 16

_mesh = plsc.VectorSubcoreMesh(core_axis_name="c", subcore_axis_name="s")


_YROWS_PER_TILE = N_NODES // NS


def _sc_scatter_body(ya_hbm, yb_hbm, eidx_hbm, p_hbm,
                     eidxv, rows0, rows1, rows2, srcj, dstj, yspm, acc,
                     g0, g1, g2, s0, s1, s2, zbuf):
    gsem = (g0, g1, g2)
    ssem = (s0, s1, s2)
    c = lax.axis_index("c")
    s = lax.axis_index("s")
    base = s * ROWS_PER_TILE

    zero16 = jnp.zeros((16,), jnp.float32)
    for i in range(zbuf.shape[0]):
        for j in range(DH // 16):
            zbuf[i, pl.ds(j * 16, 16)] = zero16

    pltpu.async_copy(eidx_hbm.at[pl.ds(s * SCAT_CHUNKS, SCAT_CHUNKS)], eidxv, g0)
    ybase = s * _YROWS_PER_TILE

    @pl.when(c == 0)
    def _():
        pltpu.async_copy(ya_hbm.at[pl.ds(ybase, _YROWS_PER_TILE)],
                         yspm.at[pl.ds(ybase, _YROWS_PER_TILE)], g1)

    @pl.when(c == 1)
    def _():
        pltpu.async_copy(yb_hbm.at[pl.ds(ybase, _YROWS_PER_TILE)],
                         yspm.at[pl.ds(ybase, _YROWS_PER_TILE)], g1)

    for t in range(ROWS_PER_TILE // zbuf.shape[0]):
        pltpu.async_copy(zbuf, acc.at[pl.ds(base + t * zbuf.shape[0],
                                            zbuf.shape[0])], g2)

    def unpack(j, b):
        for k in range(CHUNK // 16):
            v = eidxv[j, pl.ds(k * 16, 16)]
            srcj[b, pl.ds(k * 16, 16)] = lax.bitwise_and(v, 0xFFFF)
            dstj[b, pl.ds(k * 16, 16)] = lax.shift_right_logical(v, 16)

    pltpu.make_async_copy(eidx_hbm.at[pl.ds(s * SCAT_CHUNKS, SCAT_CHUNKS)],
                          eidxv, g0).wait()
    for b in range(3):
        unpack(b, b)

    @pl.when(c == 0)
    def _():
        pltpu.make_async_copy(ya_hbm.at[pl.ds(ybase, _YROWS_PER_TILE)],
                              yspm.at[pl.ds(ybase, _YROWS_PER_TILE)], g1).wait()

    @pl.when(c == 1)
    def _():
        pltpu.make_async_copy(yb_hbm.at[pl.ds(ybase, _YROWS_PER_TILE)],
                              yspm.at[pl.ds(ybase, _YROWS_PER_TILE)], g1).wait()

    for t in range(ROWS_PER_TILE // zbuf.shape[0]):
        pltpu.make_async_copy(zbuf, acc.at[pl.ds(base + t * zbuf.shape[0],
                                                 zbuf.shape[0])], g2).wait()
    plsc.subcore_barrier()

    rows = (rows0, rows1, rows2)

    def gather(j, b):
        pltpu.async_copy(yspm.at[srcj.at[b]], rows[b], gsem[b])

    def wait_gather(b):
        pltpu.make_async_copy(yspm.at[srcj.at[b]], rows[b], gsem[b]).wait()

    def scatter(b):
        pltpu.async_copy(rows[b], acc.at[dstj.at[b]], ssem[b], add=True)

    def wait_scatter(b):
        pltpu.make_async_copy(rows[b], acc.at[dstj.at[b]], ssem[b]).wait()

    for b in range(2):
        gather(b, b)
    wait_gather(0)
    scatter(0)
    gather(2, 2)

    def step3(j, b):
        wait_gather(b)
        scatter(b)
        nb = (b + 2) % 3
        wait_scatter(nb)
        unpack(j + 2, nb)
        gather(j + 2, nb)

    step3(1, 1)

    def step(i, carry):
        for b in range(3):
            step3(i * 3 + 2 + b, (2 + b) % 3)
        return carry

    lax.fori_loop(0, (SCAT_CHUNKS - 4) // 3, step, 0)
    for j in (SCAT_CHUNKS - 2, SCAT_CHUNKS - 1):
        b = j % 3
        wait_gather(b)
        scatter(b)
    for j in range(SCAT_CHUNKS - 3, SCAT_CHUNKS):
        wait_scatter(j % 3)

    plsc.subcore_barrier()
    pltpu.sync_copy(acc.at[pl.ds(base, ROWS_PER_TILE)],
                    p_hbm.at[c, pl.ds(base, ROWS_PER_TILE)])


_sc_scatter = pl.kernel(
    _sc_scatter_body,
    out_type=[jax.ShapeDtypeStruct((NC, ACC_ROWS, DH), jnp.float32)],
    mesh=_mesh,
    scratch_types=(
        [pltpu.VMEM((SCAT_CHUNKS, CHUNK), jnp.int32)]
        + [pltpu.VMEM((CHUNK, DH), jnp.float32)] * 3
        + [pltpu.VMEM((3, CHUNK), jnp.int32)] * 2
        + [pltpu.VMEM_SHARED((N_NODES, DH), jnp.float32)]
        + [pltpu.VMEM_SHARED((ACC_ROWS, DH), jnp.float32)]
        + [pltpu.SemaphoreType.DMA] * 6
        + [pltpu.VMEM((16, DH), jnp.float32)]
    ),
    compiler_params=pltpu.CompilerParams(use_tc_tiling_on_sc=False),
)


def _sc_counts_body(eidx_hbm, cnt_hbm, dstv, onesv, cacc, zcnt):
    c = lax.axis_index("c")
    s = lax.axis_index("s")
    wid = s * NC + c
    base = s * ROWS_PER_TILE

    zero16 = jnp.zeros((16,), jnp.float32)
    one16 = jnp.ones((16,), jnp.float32)
    for i in range(CHUNK):
        onesv[i, :] = one16
    for i in range(zcnt.shape[0]):
        zcnt[i, :] = zero16

    pltpu.sync_copy(eidx_hbm.at[pl.ds(wid * CNT_CHUNKS, CNT_CHUNKS)], dstv)

    def unpack(i, carry):
        for k in range(CHUNK // 16):
            dstv[i, pl.ds(k * 16, 16)] = lax.shift_right_logical(
                dstv[i, pl.ds(k * 16, 16)], 16)
        return carry

    lax.fori_loop(0, CNT_CHUNKS, unpack, 0)
    for t in range(ROWS_PER_TILE // zcnt.shape[0]):
        pltpu.sync_copy(zcnt, cacc.at[pl.ds(base + t * zcnt.shape[0], zcnt.shape[0])])
    plsc.subcore_barrier()

    def step(j, carry):
        pltpu.sync_copy(onesv, cacc.at[dstv.at[j]], add=True)
        return carry

    lax.fori_loop(0, CNT_CHUNKS, step, 0)
    plsc.subcore_barrier()

    pltpu.sync_copy(cacc.at[pl.ds(base, ROWS_PER_TILE)],
                    cnt_hbm.at[c, pl.ds(base, ROWS_PER_TILE)])


_sc_counts = pl.kernel(
    _sc_counts_body,
    out_type=[jax.ShapeDtypeStruct((NC, ACC_ROWS, CNT_W), jnp.float32)],
    mesh=_mesh,
    scratch_types=[
        pltpu.VMEM((CNT_CHUNKS, CHUNK), jnp.int32),
        pltpu.VMEM((CHUNK, CNT_W), jnp.float32),
        pltpu.VMEM_SHARED((ACC_ROWS, CNT_W), jnp.float32),
        pltpu.VMEM((64, CNT_W), jnp.float32),
    ],
    compiler_params=pltpu.CompilerParams(use_tc_tiling_on_sc=False),
)


_BR = 1000
_GRID = (N_NODES // _BR,)


def _row_spec(w=D):
    return pl.BlockSpec((_BR, w), lambda i: (i, 0))


def _plane_spec(plane, w):
    return pl.BlockSpec((1, _BR, w), lambda i, p=plane: (p, i, 0))


def _full_spec(shape):
    return pl.BlockSpec(shape, lambda i: (0,) * len(shape))


def _tc1_body(x_ref, wl_ref, wr_ref, b_ref, ya_ref, yb_ref, z_ref):
    x = x_ref[...]
    y = jnp.dot(x, wl_ref[...], preferred_element_type=jnp.float32)
    ya_ref[...] = y[:, :DH]
    yb_ref[...] = y[:, DH:]
    z_ref[...] = jnp.dot(x, wr_ref[...], preferred_element_type=jnp.float32) + b_ref[...]


def _tc2_body(pa_ref, pb_ref, ca_ref, cb_ref, z1_ref, wl_ref, wr_ref, b_ref,
              ya_ref, yb_ref, z_ref):
    inv = 1.0 / jnp.clip(ca_ref[0][:, 0:1] + cb_ref[0][:, 0:1], 1.0, None)
    mean = jnp.concatenate([pa_ref[0], pb_ref[0]], axis=1) * inv
    h = jnp.maximum(mean + z1_ref[...], 0.0)
    y = jnp.dot(h, wl_ref[...], preferred_element_type=jnp.float32)
    ya_ref[...] = y[:, :DH]
    yb_ref[...] = y[:, DH:]
    z_ref[...] = jnp.dot(h, wr_ref[...], preferred_element_type=jnp.float32) + b_ref[...]


def _tc3_body(pa_ref, pb_ref, ca_ref, cb_ref, z2_ref, o_ref):
    inv = 1.0 / jnp.clip(ca_ref[0][:, 0:1] + cb_ref[0][:, 0:1], 1.0, None)
    mean = jnp.concatenate([pa_ref[0], pb_ref[0]], axis=1) * inv
    o_ref[...] = mean + z2_ref[...]


_tc1 = pl.pallas_call(
    _tc1_body,
    grid=_GRID,
    in_specs=[_row_spec(), _full_spec((D, D)), _full_spec((D, D)),
              _full_spec((1, D))],
    out_specs=[_row_spec(DH), _row_spec(DH), _row_spec()],
    out_shape=[jax.ShapeDtypeStruct((N_NODES, DH), jnp.float32),
               jax.ShapeDtypeStruct((N_NODES, DH), jnp.float32),
               jax.ShapeDtypeStruct((N_NODES, D), jnp.float32)],
)

_tc2 = pl.pallas_call(
    _tc2_body,
    grid=_GRID,
    in_specs=[_plane_spec(0, DH), _plane_spec(1, DH), _plane_spec(0, CNT_W),
              _plane_spec(1, CNT_W), _row_spec(), _full_spec((D, D)),
              _full_spec((D, D)), _full_spec((1, D))],
    out_specs=[_row_spec(DH), _row_spec(DH), _row_spec()],
    out_shape=[jax.ShapeDtypeStruct((N_NODES, DH), jnp.float32),
               jax.ShapeDtypeStruct((N_NODES, DH), jnp.float32),
               jax.ShapeDtypeStruct((N_NODES, D), jnp.float32)],
)

_tc3 = pl.pallas_call(
    _tc3_body,
    grid=_GRID,
    in_specs=[_plane_spec(0, DH), _plane_spec(1, DH), _plane_spec(0, CNT_W),
              _plane_spec(1, CNT_W), _row_spec()],
    out_specs=_row_spec(),
    out_shape=jax.ShapeDtypeStruct((N_NODES, D), jnp.float32),
)


@jax.jit
def kernel(x, edge_index, W1_l, b1, W1_r, W2_l, b2, W2_r):
    n_edges = edge_index.shape[1]
    pad = EDGES_PAD - n_edges
    src = jnp.concatenate(
        [edge_index[0].astype(jnp.int32), jnp.zeros((pad,), jnp.int32)])
    dst = jnp.concatenate(
        [edge_index[1].astype(jnp.int32), jnp.full((pad,), DUMMY, jnp.int32)])
    eidx = (src | (dst << 16)).reshape(-1, CHUNK)

    (cnt,) = _sc_counts(eidx)
    ya1, yb1, z1 = _tc1(x, W1_l.T, W1_r.T, b1[None, :])
    (p1,) = _sc_scatter(ya1, yb1, eidx)
    ya2, yb2, z2 = _tc2(p1, p1, cnt, cnt, z1, W2_l.T, W2_r.T, b2[None, :])
    (p2,) = _sc_scatter(ya2, yb2, eidx)
    out = _tc3(p2, p2, cnt, cnt, z2)
    return out
